# baseline (speedup 1.0000x reference)
import jax
import jax.numpy as jnp
from jax.experimental import pallas as pl
from jax.experimental.pallas import tpu as pltpu

_NUM_TIMESTEPS = 1000
_BETA_START = 1e-4
_BETA_END = 0.02
_LANES = 1024


def _fma_body(t_ref, a_tab_ref, b_tab_ref, x_ref, n_ref, o_ref):
    ti = t_ref[pl.program_id(0)]
    a = a_tab_ref[ti]
    b = b_tab_ref[ti]
    o_ref[...] = a * x_ref[...] + b * n_ref[...]


def kernel(x_start, t, noise):
    B = x_start.shape[0]
    flat = x_start.size // B
    S = flat // _LANES
    x2 = x_start.reshape(B, S, _LANES)
    n2 = noise.reshape(B, S, _LANES)

    betas = jnp.linspace(_BETA_START, _BETA_END, _NUM_TIMESTEPS, dtype=jnp.float32)
    ac = jnp.cumprod(1.0 - betas, axis=0)
    a_tab = jnp.sqrt(ac)
    b_tab = jnp.sqrt(1.0 - ac)

    K = 4
    blk = S // K

    out = pl.pallas_call(
        _fma_body,
        grid=(B, K),
        in_specs=[
            pl.BlockSpec(memory_space=pltpu.SMEM),
            pl.BlockSpec(memory_space=pltpu.SMEM),
            pl.BlockSpec(memory_space=pltpu.SMEM),
            pl.BlockSpec((1, blk, _LANES), lambda i, j: (i, j, 0)),
            pl.BlockSpec((1, blk, _LANES), lambda i, j: (i, j, 0)),
        ],
        out_specs=pl.BlockSpec((1, blk, _LANES), lambda i, j: (i, j, 0)),
        out_shape=jax.ShapeDtypeStruct((B, S, _LANES), jnp.float32),
    )(t, a_tab, b_tab, x2, n2)
    return out.reshape(x_start.shape)


# speedup vs baseline: 4.7920x; 4.7920x over previous
import jax
import jax.numpy as jnp
from jax.experimental import pallas as pl
from jax.experimental.pallas import tpu as pltpu

_NUM_TIMESTEPS = 1000
_BETA_START = 1e-4
_BETA_END = 0.02
_LANES = 1024


def _fma_body(t_ref, a_tab_ref, b_tab_ref, x_ref, n_ref, o_ref):
    ti = t_ref[pl.program_id(0)]
    a = a_tab_ref[ti]
    b = b_tab_ref[ti]
    o_ref[...] = a * x_ref[...] + b * n_ref[...]


def kernel(x_start, t, noise):
    B, C, H, W = x_start.shape

    betas = jnp.linspace(_BETA_START, _BETA_END, _NUM_TIMESTEPS, dtype=jnp.float32)
    ac = jnp.cumprod(1.0 - betas, axis=0)
    a_tab = jnp.sqrt(ac)
    b_tab = jnp.sqrt(1.0 - ac)

    blk = (1, C, H, W)
    out = pl.pallas_call(
        _fma_body,
        grid=(B,),
        in_specs=[
            pl.BlockSpec(memory_space=pltpu.SMEM),
            pl.BlockSpec(memory_space=pltpu.SMEM),
            pl.BlockSpec(memory_space=pltpu.SMEM),
            pl.BlockSpec(blk, lambda i: (i, 0, 0, 0)),
            pl.BlockSpec(blk, lambda i: (i, 0, 0, 0)),
        ],
        out_specs=pl.BlockSpec(blk, lambda i: (i, 0, 0, 0)),
        out_shape=jax.ShapeDtypeStruct((B, C, H, W), jnp.float32),
    )(t, a_tab, b_tab, x_start, noise)
    return out
